# baseline (speedup 1.0000x reference)
import functools

import jax
import jax.numpy as jnp
from jax import lax
from jax.experimental import pallas as pl
from jax.experimental.pallas import tpu as pltpu
from jax.experimental.pallas import tpu_sc as plsc

_N = 4096
_T = 200
_EMB = 64
_NC = 2
_NS = 16
_NW = _NC * _NS
_SPW = _N // _NW
_HALF = _T // 2


def _pooled_sc(toks2d, tok_emb):
    mesh = plsc.VectorSubcoreMesh(core_axis_name="c", subcore_axis_name="s")

    @functools.partial(
        pl.kernel,
        out_type=jax.ShapeDtypeStruct((_N, _EMB), jnp.float32),
        mesh=mesh,
        scratch_types=[
            pltpu.VMEM((2 * _SPW, _HALF), jnp.int32),
            pltpu.VMEM((_HALF, _EMB), jnp.float32),
            pltpu.VMEM((_HALF, _EMB), jnp.float32),
            pltpu.VMEM((_SPW, _EMB), jnp.float32),
            pltpu.SemaphoreType.DMA,
            pltpu.SemaphoreType.DMA,
        ],
        compiler_params=pltpu.CompilerParams(use_tc_tiling_on_sc=False),
    )
    def k(toks_hbm, table_hbm, out_hbm, idx_v, rows0, rows1, pool_v, sem0, sem1):
        wid = lax.axis_index("s") * _NC + lax.axis_index("c")
        base = wid * _SPW
        pltpu.sync_copy(toks_hbm.at[pl.ds(2 * base, 2 * _SPW)], idx_v)
        pltpu.async_copy(table_hbm.at[idx_v.at[0]], rows0, sem0)

        def accum(rows, accs):
            def body(i, accs):
                for dt in range(4):
                    t = i * 4 + dt
                    accs = tuple(
                        a + rows[t, pl.ds(16 * c, 16)] for c, a in enumerate(accs)
                    )
                return accs
            return lax.fori_loop(0, _HALF // 4, body, accs)

        inv_t = jnp.float32(1.0 / _T)

        def sample_body(s, carry):
            pltpu.async_copy(table_hbm.at[idx_v.at[2 * s + 1]], rows1, sem1)
            pltpu.make_async_copy(table_hbm.at[idx_v.at[2 * s]], rows0, sem0).wait()
            z = jnp.zeros((16,), jnp.float32)
            accs = accum(rows0, (z, z, z, z))
            nxt = (2 * s + 2) & (2 * _SPW - 1)
            pltpu.async_copy(table_hbm.at[idx_v.at[nxt]], rows0, sem0)
            pltpu.make_async_copy(table_hbm.at[idx_v.at[2 * s + 1]], rows1, sem1).wait()
            accs = accum(rows1, accs)
            for c, a in enumerate(accs):
                pool_v[s, pl.ds(16 * c, 16)] = a * inv_t
            return carry

        lax.fori_loop(0, _SPW, sample_body, 0)
        pltpu.make_async_copy(table_hbm.at[idx_v.at[0]], rows0, sem0).wait()
        pltpu.sync_copy(pool_v, out_hbm.at[pl.ds(base, _SPW)])

    return k(toks2d, tok_emb)


def _head_tc(pooled, W, b2, stdev11, noise):
    def body(p_ref, w_ref, b_ref, s_ref, n_ref, o_ref):
        p = p_ref[...]
        w = w_ref[...]
        logits = lax.dot_general(
            p, w, (((1,), (0,)), ((), ())), preferred_element_type=jnp.float32
        )
        x = logits + b_ref[...] + n_ref[...] * s_ref[0, 0]
        m = jnp.max(x, axis=-1, keepdims=True)
        e = jnp.exp(x - m)
        o_ref[...] = (x - m) - jnp.log(jnp.sum(e, axis=-1, keepdims=True))

    return pl.pallas_call(
        body,
        out_shape=jax.ShapeDtypeStruct((_N, 2), jnp.float32),
    )(pooled, W, b2, stdev11, noise)


def kernel(toks, tok_emb, W, b, stdev):
    toks2d = toks.reshape(2 * _N, _HALF)
    pooled = _pooled_sc(toks2d, tok_emb)
    noise = jax.random.normal(jax.random.key(1234), (_N, 2), dtype=jnp.float32)
    return _head_tc(pooled, W, b.reshape(1, 2), stdev.reshape(1, 1), noise)


# speedup vs baseline: 1.0879x; 1.0879x over previous
import functools

import jax
import jax.numpy as jnp
from jax import lax
from jax.experimental import pallas as pl
from jax.experimental.pallas import tpu as pltpu
from jax.experimental.pallas import tpu_sc as plsc

_N = 4096
_T = 200
_EMB = 64
_NC = 2
_NS = 16
_NW = _NC * _NS
_SPW = _N // _NW
_C0 = 96
_C1 = _T - _C0
_CNT = (_C0, _C1)
_NBUF = 4


def _pooled_sc(toks, tok_emb):
    mesh = plsc.VectorSubcoreMesh(core_axis_name="c", subcore_axis_name="s")

    @functools.partial(
        pl.kernel,
        out_type=jax.ShapeDtypeStruct((_N, _EMB), jnp.float32),
        mesh=mesh,
        scratch_types=[
            pltpu.VMEM((_SPW, _T), jnp.int32),
            [pltpu.VMEM((_C1, _EMB), jnp.float32) for _ in range(_NBUF)],
            pltpu.VMEM((_SPW, _EMB), jnp.float32),
            [pltpu.SemaphoreType.DMA for _ in range(_NBUF)],
        ],
        compiler_params=pltpu.CompilerParams(use_tc_tiling_on_sc=False),
    )
    def k(toks_hbm, table_hbm, out_hbm, idx_v, rows, pool_v, sems):
        wid = lax.axis_index("s") * _NC + lax.axis_index("c")
        base = wid * _SPW
        pltpu.sync_copy(toks_hbm.at[pl.ds(base, _SPW)], idx_v)

        def start(s, j, buf):
            pltpu.async_copy(
                table_hbm.at[idx_v.at[s, pl.ds(j * _C0, _CNT[j])]],
                rows[buf].at[pl.ds(0, _CNT[j])],
                sems[buf],
            )

        def wait(s, j, buf):
            pltpu.make_async_copy(
                table_hbm.at[idx_v.at[s, pl.ds(j * _C0, _CNT[j])]],
                rows[buf].at[pl.ds(0, _CNT[j])],
                sems[buf],
            ).wait()

        start(0, 0, 0)
        start(0, 1, 1)
        start(1, 0, 2)

        def accum(ref, cnt, accs):
            def body(i, accs):
                for dt in range(4):
                    t = i * 4 + dt
                    accs = tuple(
                        a + ref[t, pl.ds(16 * c, 16)] for c, a in enumerate(accs)
                    )
                return accs
            return lax.fori_loop(0, cnt // 4, body, accs)

        inv_t = jnp.float32(1.0 / _T)
        smask = _SPW - 1

        def do_phase(s, phase):
            z = jnp.zeros((16,), jnp.float32)
            accs = (z, z, z, z)
            for j in range(2):
                buf = (2 * phase + j) % _NBUF
                nxt = (buf + _NBUF - 1) % _NBUF
                if j == 0:
                    start((s + 1) & smask, 1, nxt)
                else:
                    start((s + 2) & smask, 0, nxt)
                wait(s, j, buf)
                accs = accum(rows[buf], _CNT[j], accs)
            for c, a in enumerate(accs):
                pool_v[s, pl.ds(16 * c, 16)] = a * inv_t

        def sample_body(s, carry):
            @pl.when(lax.rem(s, 2) == 0)
            def _():
                do_phase(s, 0)

            @pl.when(lax.rem(s, 2) == 1)
            def _():
                do_phase(s, 1)

            return carry

        lax.fori_loop(0, _SPW, sample_body, 0)
        wait(0, 0, 0)
        wait(0, 1, 1)
        wait(1, 0, 2)
        pltpu.sync_copy(pool_v, out_hbm.at[pl.ds(base, _SPW)])

    return k(toks, tok_emb)


def _head_tc(pooled, W, b2, stdev11, noise):
    def body(p_ref, w_ref, b_ref, s_ref, n_ref, o_ref):
        p = p_ref[...]
        w = w_ref[...]
        logits = lax.dot_general(
            p, w, (((1,), (0,)), ((), ())), preferred_element_type=jnp.float32
        )
        x = logits + b_ref[...] + n_ref[...] * s_ref[0, 0]
        m = jnp.max(x, axis=-1, keepdims=True)
        e = jnp.exp(x - m)
        o_ref[...] = (x - m) - jnp.log(jnp.sum(e, axis=-1, keepdims=True))

    return pl.pallas_call(
        body,
        out_shape=jax.ShapeDtypeStruct((_N, 2), jnp.float32),
    )(pooled, W, b2, stdev11, noise)


def kernel(toks, tok_emb, W, b, stdev):
    pooled = _pooled_sc(toks, tok_emb)
    noise = jax.random.normal(jax.random.key(1234), (_N, 2), dtype=jnp.float32)
    return _head_tc(pooled, W, b.reshape(1, 2), stdev.reshape(1, 1), noise)
